# baseline (speedup 1.0000x reference)
import functools

import jax
import jax.numpy as jnp
from jax import lax
from jax.experimental import pallas as pl
from jax.experimental.pallas import tpu as pltpu
from jax.experimental.pallas import tpu_sc as plsc

N = 50000
E = 800000
D = 8
NP = 51200
NC = 2
NS = 16

_F32 = jnp.float32
_I32 = jnp.int32


def _dyn_gather(v, idx):
  dnums = lax.GatherDimensionNumbers(
      offset_dims=(), collapsed_slice_dims=(0,), start_index_map=(0,))
  return lax.gather(v, idx[:, None], dnums, (1,),
                    mode=lax.GatherScatterMode.PROMISE_IN_BOUNDS)


def _combine(is_min_vec, a, b):
  return jnp.where(is_min_vec, jnp.minimum(a, b), jnp.maximum(a, b))


_HASH = 8192
_LC = 256


def _slow_sort_vec(src_buf, dst_buf, src_arr, acc, j, is_min_vec, iota,
                   shift_idx, next_idx, last_mask):
  s = src_buf[pl.ds(j * 16, 16)]
  d = dst_buf[pl.ds(j * 16, 16)]
  v = plsc.load_gather(src_arr, [s])
  res = plsc.sort_key_val(d, v)
  d_s, v_s = res[0], res[1]
  for t, pidx in enumerate(shift_idx):
    pk = _dyn_gather(d_s, pidx)
    pv = _dyn_gather(v_s, pidx)
    ok = (pk == d_s) & (iota >= (1 << t))
    v_s = jnp.where(ok, _combine(is_min_vec, v_s, pv), v_s)
  nk = _dyn_gather(d_s, next_idx)
  is_end = (d_s != nk) | last_mask
  cur = plsc.load_gather(acc, [d_s])
  plsc.store_scatter(acc, [d_s], _combine(is_min_vec, cur, v_s), mask=is_end)


def _edge_scatter_body(src_vals_h, src_h, dst_h, out_h, val_v, acc, sb, db,
                       hsc, lc, *, n_streams, e_slice, ch, row_of, vrow_of):
  wid = lax.axis_index("s") * NC + lax.axis_index("c")
  kind = (wid // n_streams) % 2
  is_min_vec = (jnp.zeros((16,), _I32) + kind) == 0
  iota = lax.iota(_I32, 16)
  shift_idx = tuple(jnp.maximum(iota - (1 << t), 0) for t in range(4))
  next_idx = jnp.minimum(iota + 1, 15)
  last_mask = iota == 15
  lane0 = iota == 0
  nv = ch // 16
  gnv = (nv + 15) // 16

  voff = pl.multiple_of(vrow_of(wid) * N, 8)
  pltpu.sync_copy(src_vals_h.at[pl.ds(voff, N)], val_v)
  pltpu.sync_copy(src_vals_h.at[pl.ds(voff, N)], acc)
  lc[pl.ds(16 * (gnv - 1), 16)] = jnp.zeros((16,), _I32)

  slice_id = wid // (2 * n_streams)
  base = slice_id * e_slice

  def chunk_body(ich, carry):
    off = pl.multiple_of(base + ich * ch, 8)
    pltpu.sync_copy(src_h.at[pl.ds(off, ch)], sb)
    pltpu.sync_copy(dst_h.at[pl.ds(off, ch)], db)

    @plsc.parallel_loop(0, nv)
    def p1(j):
      d = db[pl.ds(j * 16, 16)]
      plsc.store_scatter(hsc, [d & (_HASH - 1)], iota + j * 16)

    @plsc.parallel_loop(0, nv)
    def p2(j):
      d = db[pl.ds(j * 16, 16)]
      s = sb[pl.ds(j * 16, 16)]
      back = plsc.load_gather(hsc, [d & (_HASH - 1)])
      w = back == (iota + j * 16)
      v = plsc.load_gather(val_v, [s])
      cur = plsc.load_gather(acc, [d])
      plsc.store_scatter(acc, [d], _combine(is_min_vec, cur, v), mask=w)
      n = plsc.all_reduce_population_count(jnp.logical_not(w))
      plsc.store_scatter(lc, [jnp.zeros((16,), _I32) + j], n, mask=lane0)

    def p3(g, c3):
      dirty = lc[pl.ds(g * 16, 16)] > 0

      def do_dirty():
        def wcond(m):
          return jnp.any(m)

        def wbody(m):
          lane = plsc.all_reduce_ffs(m)[0]
          _slow_sort_vec(sb, db, val_v, acc, g * 16 + lane, is_min_vec,
                         iota, shift_idx, next_idx, last_mask)
          return m & (iota != lane)

        lax.while_loop(wcond, wbody, dirty)

      lax.cond(jnp.any(dirty), do_dirty, lambda: None)
      return c3

    lax.fori_loop(0, gnv, p3, 0)
    return carry

  lax.fori_loop(0, e_slice // ch, chunk_body, 0)
  ooff = pl.multiple_of(row_of(wid) * NP, 8)
  pltpu.sync_copy(acc, out_h.at[pl.ds(ooff, N)])


def _edge_scatter8(xt, src, dst):
  ch = 4000
  mesh = plsc.VectorSubcoreMesh(core_axis_name="c", subcore_axis_name="s")
  body = functools.partial(
      _edge_scatter_body,
      n_streams=8, e_slice=E // 2, ch=ch,
      row_of=lambda wid: (wid % 8) * 4 + ((wid // 8) % 2) * 2 + wid // 16,
      vrow_of=lambda wid: wid % 8)

  @functools.partial(
      pl.kernel, mesh=mesh,
      compiler_params=pltpu.CompilerParams(needs_layout_passes=False),
      out_type=jax.ShapeDtypeStruct((32 * NP,), _F32),
      scratch_types=[
          pltpu.VMEM((N,), _F32), pltpu.VMEM((N,), _F32),
          pltpu.VMEM((ch,), _I32), pltpu.VMEM((ch,), _I32),
          pltpu.VMEM((_HASH,), _I32), pltpu.VMEM((_LC,), _I32),
      ])
  def run(xt_h, src_h, dst_h, out_h, val_v, acc, sb, db, hsc, lc):
    body(xt_h, src_h, dst_h, out_h, val_v, acc, sb, db, hsc, lc)

  return run(xt.reshape(-1), src, dst)


def _edge_scatter1(vals, src, dst):
  ch = 2000
  mesh = plsc.VectorSubcoreMesh(core_axis_name="c", subcore_axis_name="s")
  body = functools.partial(
      _edge_scatter_body,
      n_streams=1, e_slice=E // 16, ch=ch,
      row_of=lambda wid: (wid % 2) * 16 + wid // 2,
      vrow_of=lambda wid: 0)

  @functools.partial(
      pl.kernel, mesh=mesh,
      compiler_params=pltpu.CompilerParams(needs_layout_passes=False),
      out_type=jax.ShapeDtypeStruct((32 * NP,), _F32),
      scratch_types=[
          pltpu.VMEM((N,), _F32), pltpu.VMEM((N,), _F32),
          pltpu.VMEM((ch,), _I32), pltpu.VMEM((ch,), _I32),
          pltpu.VMEM((_HASH,), _I32), pltpu.VMEM((_LC,), _I32),
      ])
  def run(vals_h, src_h, dst_h, out_h, val_v, acc, sb, db, hsc, lc):
    body(vals_h, src_h, dst_h, out_h, val_v, acc, sb, db, hsc, lc)

  return run(vals, src, dst)


_B1 = 2048


def _t1_body(p_ref, x_ref, w_ref, b_ref, o_ref):
  acc = jnp.zeros((_B1,), _F32)
  for c in range(D):
    mn = jnp.minimum(p_ref[4 * c + 0], p_ref[4 * c + 1])
    mx = jnp.maximum(p_ref[4 * c + 2], p_ref[4 * c + 3])
    xc = x_ref[c]
    m = jnp.maximum(xc - mn, mx - xc)
    acc = acc + m * w_ref[c, 0]
  o_ref[...] = jnp.maximum(acc + b_ref[0], 0.0)


def _t1(p1, xt_p, W1, b1):
  return pl.pallas_call(
      _t1_body,
      grid=(NP // _B1,),
      in_specs=[
          pl.BlockSpec((32, _B1), lambda i: (0, i)),
          pl.BlockSpec((D, _B1), lambda i: (0, i)),
          pl.BlockSpec(memory_space=pltpu.SMEM),
          pl.BlockSpec(memory_space=pltpu.SMEM),
      ],
      out_specs=pl.BlockSpec((_B1,), lambda i: (i,)),
      out_shape=jax.ShapeDtypeStruct((NP,), _F32),
  )(p1, xt_p, W1, b1)


def _t2_body(p_ref, a_ref, o_ref):
  mn = jnp.min(p_ref[0:16], axis=0)
  mx = jnp.max(p_ref[16:32], axis=0)
  a = a_ref[...]
  o_ref[...] = jnp.maximum(a - mn, mx - a)


def _t2(p2, a):
  return pl.pallas_call(
      _t2_body,
      grid=(NP // _B1,),
      in_specs=[
          pl.BlockSpec((32, _B1), lambda i: (0, i)),
          pl.BlockSpec((_B1,), lambda i: (i,)),
      ],
      out_specs=pl.BlockSpec((_B1,), lambda i: (i,)),
      out_shape=jax.ShapeDtypeStruct((NP,), _F32),
  )(p2, a)


_B3 = 1024


def _t3_body(p_ref, t_ref, w2_ref, b2_ref, w3_ref, b3_ref, o_ref):
  tn = jnp.min(p_ref[0:16], axis=0)
  tx = jnp.max(p_ref[16:32], axis=0)
  t = t_ref[...]
  w2 = w2_ref[0]
  b2 = b2_ref[0]
  w3 = w3_ref[0]

  def g(u):
    return jnp.maximum(u[:, None] * w2[None, :] + b2[None, :], 0.0)

  ht = g(t)
  m3 = jnp.maximum(jnp.abs(ht - g(tn)), jnp.abs(ht - g(tx)))
  z = jnp.sum(m3 * w3[None, :], axis=1) + b3_ref[0]
  o_ref[...] = jax.nn.sigmoid(z)


def _t3(p3, m2, W2, b2, W3, b3):
  return pl.pallas_call(
      _t3_body,
      grid=(NP // _B3,),
      in_specs=[
          pl.BlockSpec((32, _B3), lambda i: (0, i)),
          pl.BlockSpec((_B3,), lambda i: (i,)),
          pl.BlockSpec((1, 64), lambda i: (0, 0)),
          pl.BlockSpec((1, 64), lambda i: (0, 0)),
          pl.BlockSpec((1, 64), lambda i: (0, 0)),
          pl.BlockSpec(memory_space=pltpu.SMEM),
      ],
      out_specs=pl.BlockSpec((_B3,), lambda i: (i,)),
      out_shape=jax.ShapeDtypeStruct((NP,), _F32),
  )(p3, m2, W2, b2.reshape(1, 64), W3.reshape(1, 64), b3)


def kernel(x, edge_index, W1, b1, W2, b2, W3, b3):
  xt = x.T
  xt_p = jnp.pad(xt, ((0, 0), (0, NP - N)))
  src = edge_index[0]
  dst = edge_index[1]

  p1 = _edge_scatter8(xt, src, dst).reshape(32, NP)
  a = _t1(p1, xt_p, W1, b1)
  p2 = _edge_scatter1(a, src, dst).reshape(32, NP)
  m2 = _t2(p2, a)
  p3 = _edge_scatter1(m2, src, dst).reshape(32, NP)
  out = _t3(p3, m2, W2, b2, W3, b3)
  return out[:N].reshape(N, 1)


# speedup vs baseline: 1.8071x; 1.8071x over previous
import functools

import jax
import jax.numpy as jnp
from jax import lax
from jax.experimental import pallas as pl
from jax.experimental.pallas import tpu as pltpu
from jax.experimental.pallas import tpu_sc as plsc

N = 50000
E = 800000
D = 8
NP = 51200
NC = 2
NS = 16

_F32 = jnp.float32
_I32 = jnp.int32


def _dyn_gather(v, idx):
  dnums = lax.GatherDimensionNumbers(
      offset_dims=(), collapsed_slice_dims=(0,), start_index_map=(0,))
  return lax.gather(v, idx[:, None], dnums, (1,),
                    mode=lax.GatherScatterMode.PROMISE_IN_BOUNDS)


def _combine(is_min_vec, a, b):
  return jnp.where(is_min_vec, jnp.minimum(a, b), jnp.maximum(a, b))


def _slow_sort_vec(src_buf, dst_buf, src_arr, acc, j, is_min_vec, iota,
                   shift_idx, next_idx, last_mask):
  s = src_buf[pl.ds(j * 16, 16)]
  d = dst_buf[pl.ds(j * 16, 16)]
  v = plsc.load_gather(src_arr, [s])
  res = plsc.sort_key_val(d, v)
  d_s, v_s = res[0], res[1]
  for t, pidx in enumerate(shift_idx):
    pk = _dyn_gather(d_s, pidx)
    pv = _dyn_gather(v_s, pidx)
    ok = (pk == d_s) & (iota >= (1 << t))
    v_s = jnp.where(ok, _combine(is_min_vec, v_s, pv), v_s)
  nk = _dyn_gather(d_s, next_idx)
  is_end = (d_s != nk) | last_mask
  cur = plsc.load_gather(acc, [d_s])
  plsc.store_scatter(acc, [d_s], _combine(is_min_vec, cur, v_s), mask=is_end)


def _edge_scatter_body(src_vals_h, src_h, dst_h, out_h, val_v, acc, sb, db,
                       *, n_streams, e_slice, ch, row_of, vrow_of):
  wid = lax.axis_index("s") * NC + lax.axis_index("c")
  kind = (wid // n_streams) % 2
  is_min_vec = (jnp.zeros((16,), _I32) + kind) == 0
  iota = lax.iota(_I32, 16)
  shift_idx = tuple(jnp.maximum(iota - (1 << t), 0) for t in range(4))
  next_idx = jnp.minimum(iota + 1, 15)
  last_mask = iota == 15
  nv = ch // 16

  voff = pl.multiple_of(vrow_of(wid) * N, 8)
  pltpu.sync_copy(src_vals_h.at[pl.ds(voff, N)], val_v)
  pltpu.sync_copy(src_vals_h.at[pl.ds(voff, N)], acc)

  slice_id = wid // (2 * n_streams)
  base = slice_id * e_slice

  def chunk_body(ich, carry):
    off = pl.multiple_of(base + ich * ch, 8)
    pltpu.sync_copy(src_h.at[pl.ds(off, ch)], sb)
    pltpu.sync_copy(dst_h.at[pl.ds(off, ch)], db)

    def vec_body(j, bad):
      d = db[pl.ds(j * 16, 16)]
      s = sb[pl.ds(j * 16, 16)]
      v = plsc.load_gather(val_v, [s])
      cur = plsc.load_gather(acc, [d])
      plsc.store_scatter(acc, [d], _combine(is_min_vec, cur, v))
      back = plsc.load_gather(acc, [d])
      need = jnp.where(is_min_vec, back > v, back < v)
      plsc.store_scatter(acc, [d], v, mask=need)
      back2 = plsc.load_gather(acc, [d])
      return bad | jnp.where(is_min_vec, back2 > v, back2 < v)

    bad = lax.fori_loop(0, nv, vec_body, jnp.zeros((16,), jnp.bool_))

    def cleanup():
      def cb(j, c):
        _slow_sort_vec(sb, db, val_v, acc, j, is_min_vec, iota,
                       shift_idx, next_idx, last_mask)
        return c

      lax.fori_loop(0, nv, cb, 0)

    lax.cond(jnp.any(bad), cleanup, lambda: None)
    return carry

  lax.fori_loop(0, e_slice // ch, chunk_body, 0)
  ooff = pl.multiple_of(row_of(wid) * NP, 8)
  pltpu.sync_copy(acc, out_h.at[pl.ds(ooff, N)])


def _edge_scatter8(xt, src, dst):
  ch = 4000
  mesh = plsc.VectorSubcoreMesh(core_axis_name="c", subcore_axis_name="s")
  body = functools.partial(
      _edge_scatter_body,
      n_streams=8, e_slice=E // 2, ch=ch,
      row_of=lambda wid: (wid % 8) * 4 + ((wid // 8) % 2) * 2 + wid // 16,
      vrow_of=lambda wid: wid % 8)

  @functools.partial(
      pl.kernel, mesh=mesh,
      compiler_params=pltpu.CompilerParams(needs_layout_passes=False),
      out_type=jax.ShapeDtypeStruct((32 * NP,), _F32),
      scratch_types=[
          pltpu.VMEM((N,), _F32), pltpu.VMEM((N,), _F32),
          pltpu.VMEM((ch,), _I32), pltpu.VMEM((ch,), _I32),
      ])
  def run(xt_h, src_h, dst_h, out_h, val_v, acc, sb, db):
    body(xt_h, src_h, dst_h, out_h, val_v, acc, sb, db)

  return run(xt.reshape(-1), src, dst)


def _edge_scatter1(vals, src, dst):
  ch = 2000
  mesh = plsc.VectorSubcoreMesh(core_axis_name="c", subcore_axis_name="s")
  body = functools.partial(
      _edge_scatter_body,
      n_streams=1, e_slice=E // 16, ch=ch,
      row_of=lambda wid: (wid % 2) * 16 + wid // 2,
      vrow_of=lambda wid: 0)

  @functools.partial(
      pl.kernel, mesh=mesh,
      compiler_params=pltpu.CompilerParams(needs_layout_passes=False),
      out_type=jax.ShapeDtypeStruct((32 * NP,), _F32),
      scratch_types=[
          pltpu.VMEM((N,), _F32), pltpu.VMEM((N,), _F32),
          pltpu.VMEM((ch,), _I32), pltpu.VMEM((ch,), _I32),
      ])
  def run(vals_h, src_h, dst_h, out_h, val_v, acc, sb, db):
    body(vals_h, src_h, dst_h, out_h, val_v, acc, sb, db)

  return run(vals, src, dst)


_B1 = 2048


def _t1_body(p_ref, x_ref, w_ref, b_ref, o_ref):
  acc = jnp.zeros((_B1,), _F32)
  for c in range(D):
    mn = jnp.minimum(p_ref[4 * c + 0], p_ref[4 * c + 1])
    mx = jnp.maximum(p_ref[4 * c + 2], p_ref[4 * c + 3])
    xc = x_ref[c]
    m = jnp.maximum(xc - mn, mx - xc)
    acc = acc + m * w_ref[c, 0]
  o_ref[...] = jnp.maximum(acc + b_ref[0], 0.0)


def _t1(p1, xt_p, W1, b1):
  return pl.pallas_call(
      _t1_body,
      grid=(NP // _B1,),
      in_specs=[
          pl.BlockSpec((32, _B1), lambda i: (0, i)),
          pl.BlockSpec((D, _B1), lambda i: (0, i)),
          pl.BlockSpec(memory_space=pltpu.SMEM),
          pl.BlockSpec(memory_space=pltpu.SMEM),
      ],
      out_specs=pl.BlockSpec((_B1,), lambda i: (i,)),
      out_shape=jax.ShapeDtypeStruct((NP,), _F32),
  )(p1, xt_p, W1, b1)


def _t2_body(p_ref, a_ref, o_ref):
  mn = jnp.min(p_ref[0:16], axis=0)
  mx = jnp.max(p_ref[16:32], axis=0)
  a = a_ref[...]
  o_ref[...] = jnp.maximum(a - mn, mx - a)


def _t2(p2, a):
  return pl.pallas_call(
      _t2_body,
      grid=(NP // _B1,),
      in_specs=[
          pl.BlockSpec((32, _B1), lambda i: (0, i)),
          pl.BlockSpec((_B1,), lambda i: (i,)),
      ],
      out_specs=pl.BlockSpec((_B1,), lambda i: (i,)),
      out_shape=jax.ShapeDtypeStruct((NP,), _F32),
  )(p2, a)


_B3 = 1024


def _t3_body(p_ref, t_ref, w2_ref, b2_ref, w3_ref, b3_ref, o_ref):
  tn = jnp.min(p_ref[0:16], axis=0)
  tx = jnp.max(p_ref[16:32], axis=0)
  t = t_ref[...]
  w2 = w2_ref[0]
  b2 = b2_ref[0]
  w3 = w3_ref[0]

  def g(u):
    return jnp.maximum(u[:, None] * w2[None, :] + b2[None, :], 0.0)

  ht = g(t)
  m3 = jnp.maximum(jnp.abs(ht - g(tn)), jnp.abs(ht - g(tx)))
  z = jnp.sum(m3 * w3[None, :], axis=1) + b3_ref[0]
  o_ref[...] = jax.nn.sigmoid(z)


def _t3(p3, m2, W2, b2, W3, b3):
  return pl.pallas_call(
      _t3_body,
      grid=(NP // _B3,),
      in_specs=[
          pl.BlockSpec((32, _B3), lambda i: (0, i)),
          pl.BlockSpec((_B3,), lambda i: (i,)),
          pl.BlockSpec((1, 64), lambda i: (0, 0)),
          pl.BlockSpec((1, 64), lambda i: (0, 0)),
          pl.BlockSpec((1, 64), lambda i: (0, 0)),
          pl.BlockSpec(memory_space=pltpu.SMEM),
      ],
      out_specs=pl.BlockSpec((_B3,), lambda i: (i,)),
      out_shape=jax.ShapeDtypeStruct((NP,), _F32),
  )(p3, m2, W2, b2.reshape(1, 64), W3.reshape(1, 64), b3)


def kernel(x, edge_index, W1, b1, W2, b2, W3, b3):
  xt = x.T
  xt_p = jnp.pad(xt, ((0, 0), (0, NP - N)))
  src = edge_index[0]
  dst = edge_index[1]

  p1 = _edge_scatter8(xt, src, dst).reshape(32, NP)
  a = _t1(p1, xt_p, W1, b1)
  p2 = _edge_scatter1(a, src, dst).reshape(32, NP)
  m2 = _t2(p2, a)
  p3 = _edge_scatter1(m2, src, dst).reshape(32, NP)
  out = _t3(p3, m2, W2, b2, W3, b3)
  return out[:N].reshape(N, 1)


# speedup vs baseline: 2.9079x; 1.6092x over previous
import functools

import jax
import jax.numpy as jnp
from jax import lax
from jax.experimental import pallas as pl
from jax.experimental.pallas import tpu as pltpu
from jax.experimental.pallas import tpu_sc as plsc

N = 50000
E = 800000
D = 8
NP = 51200
NC = 2
NS = 16

_U = 5
_F32 = jnp.float32
_I32 = jnp.int32


def _dyn_gather(v, idx):
  dnums = lax.GatherDimensionNumbers(
      offset_dims=(), collapsed_slice_dims=(0,), start_index_map=(0,))
  return lax.gather(v, idx[:, None], dnums, (1,),
                    mode=lax.GatherScatterMode.PROMISE_IN_BOUNDS)


def _combine(is_min_vec, a, b):
  return jnp.where(is_min_vec, jnp.minimum(a, b), jnp.maximum(a, b))


def _slow_sort_vec(src_buf, dst_buf, src_arr, acc, j, is_min_vec, iota,
                   shift_idx, next_idx, last_mask):
  s = src_buf[pl.ds(j * 16, 16)]
  d = dst_buf[pl.ds(j * 16, 16)]
  v = plsc.load_gather(src_arr, [s])
  res = plsc.sort_key_val(d, v)
  d_s, v_s = res[0], res[1]
  for t, pidx in enumerate(shift_idx):
    pk = _dyn_gather(d_s, pidx)
    pv = _dyn_gather(v_s, pidx)
    ok = (pk == d_s) & (iota >= (1 << t))
    v_s = jnp.where(ok, _combine(is_min_vec, v_s, pv), v_s)
  nk = _dyn_gather(d_s, next_idx)
  is_end = (d_s != nk) | last_mask
  cur = plsc.load_gather(acc, [d_s])
  plsc.store_scatter(acc, [d_s], _combine(is_min_vec, cur, v_s), mask=is_end)


def _edge_scatter_body(src_vals_h, src_h, dst_h, out_h, val_v, acc, sb, db,
                       *, n_streams, e_slice, ch, row_of, vrow_of):
  wid = lax.axis_index("s") * NC + lax.axis_index("c")
  kind = (wid // n_streams) % 2
  is_min_vec = (jnp.zeros((16,), _I32) + kind) == 0
  iota = lax.iota(_I32, 16)
  shift_idx = tuple(jnp.maximum(iota - (1 << t), 0) for t in range(4))
  next_idx = jnp.minimum(iota + 1, 15)
  last_mask = iota == 15
  nv = ch // 16

  voff = pl.multiple_of(vrow_of(wid) * N, 8)
  pltpu.sync_copy(src_vals_h.at[pl.ds(voff, N)], val_v)
  pltpu.sync_copy(src_vals_h.at[pl.ds(voff, N)], acc)

  slice_id = wid // (2 * n_streams)
  base = slice_id * e_slice

  def chunk_body(ich, carry):
    off = pl.multiple_of(base + ich * ch, 8)
    pltpu.sync_copy(src_h.at[pl.ds(off, ch)], sb)
    pltpu.sync_copy(dst_h.at[pl.ds(off, ch)], db)

    def vec_body(g, bad):
      j0 = g * _U
      ds_ = [db[pl.ds((j0 + u) * 16, 16)] for u in range(_U)]
      ss_ = [sb[pl.ds((j0 + u) * 16, 16)] for u in range(_U)]
      vs_ = [plsc.load_gather(val_v, [s]) for s in ss_]
      curs = [plsc.load_gather(acc, [d]) for d in ds_]
      for d, cur, v in zip(ds_, curs, vs_):
        plsc.store_scatter(acc, [d], _combine(is_min_vec, cur, v))
      backs = [plsc.load_gather(acc, [d]) for d in ds_]
      needs = [jnp.where(is_min_vec, b > v, b < v)
               for b, v in zip(backs, vs_)]
      for d, v, need in zip(ds_, vs_, needs):
        plsc.store_scatter(acc, [d], v, mask=need)
      back2s = [plsc.load_gather(acc, [d]) for d in ds_]
      for b2, v in zip(back2s, vs_):
        bad = bad | jnp.where(is_min_vec, b2 > v, b2 < v)
      return bad

    bad = lax.fori_loop(0, nv // _U, vec_body, jnp.zeros((16,), jnp.bool_))

    def cleanup():
      def cb(j, c):
        _slow_sort_vec(sb, db, val_v, acc, j, is_min_vec, iota,
                       shift_idx, next_idx, last_mask)
        return c

      lax.fori_loop(0, nv, cb, 0)

    lax.cond(jnp.any(bad), cleanup, lambda: None)
    return carry

  lax.fori_loop(0, e_slice // ch, chunk_body, 0)
  ooff = pl.multiple_of(row_of(wid) * NP, 8)
  pltpu.sync_copy(acc, out_h.at[pl.ds(ooff, N)])


def _edge_scatter8(xt, src, dst):
  ch = 4000
  mesh = plsc.VectorSubcoreMesh(core_axis_name="c", subcore_axis_name="s")
  body = functools.partial(
      _edge_scatter_body,
      n_streams=8, e_slice=E // 2, ch=ch,
      row_of=lambda wid: (wid % 8) * 4 + ((wid // 8) % 2) * 2 + wid // 16,
      vrow_of=lambda wid: wid % 8)

  @functools.partial(
      pl.kernel, mesh=mesh,
      compiler_params=pltpu.CompilerParams(needs_layout_passes=False),
      out_type=jax.ShapeDtypeStruct((32 * NP,), _F32),
      scratch_types=[
          pltpu.VMEM((N,), _F32), pltpu.VMEM((N,), _F32),
          pltpu.VMEM((ch,), _I32), pltpu.VMEM((ch,), _I32),
      ])
  def run(xt_h, src_h, dst_h, out_h, val_v, acc, sb, db):
    body(xt_h, src_h, dst_h, out_h, val_v, acc, sb, db)

  return run(xt.reshape(-1), src, dst)


def _edge_scatter1(vals, src, dst):
  ch = 2000
  mesh = plsc.VectorSubcoreMesh(core_axis_name="c", subcore_axis_name="s")
  body = functools.partial(
      _edge_scatter_body,
      n_streams=1, e_slice=E // 16, ch=ch,
      row_of=lambda wid: (wid % 2) * 16 + wid // 2,
      vrow_of=lambda wid: 0)

  @functools.partial(
      pl.kernel, mesh=mesh,
      compiler_params=pltpu.CompilerParams(needs_layout_passes=False),
      out_type=jax.ShapeDtypeStruct((32 * NP,), _F32),
      scratch_types=[
          pltpu.VMEM((N,), _F32), pltpu.VMEM((N,), _F32),
          pltpu.VMEM((ch,), _I32), pltpu.VMEM((ch,), _I32),
      ])
  def run(vals_h, src_h, dst_h, out_h, val_v, acc, sb, db):
    body(vals_h, src_h, dst_h, out_h, val_v, acc, sb, db)

  return run(vals, src, dst)


_B1 = 2048


def _t1_body(p_ref, x_ref, w_ref, b_ref, o_ref):
  acc = jnp.zeros((_B1,), _F32)
  for c in range(D):
    mn = jnp.minimum(p_ref[4 * c + 0], p_ref[4 * c + 1])
    mx = jnp.maximum(p_ref[4 * c + 2], p_ref[4 * c + 3])
    xc = x_ref[c]
    m = jnp.maximum(xc - mn, mx - xc)
    acc = acc + m * w_ref[c, 0]
  o_ref[...] = jnp.maximum(acc + b_ref[0], 0.0)


def _t1(p1, xt_p, W1, b1):
  return pl.pallas_call(
      _t1_body,
      grid=(NP // _B1,),
      in_specs=[
          pl.BlockSpec((32, _B1), lambda i: (0, i)),
          pl.BlockSpec((D, _B1), lambda i: (0, i)),
          pl.BlockSpec(memory_space=pltpu.SMEM),
          pl.BlockSpec(memory_space=pltpu.SMEM),
      ],
      out_specs=pl.BlockSpec((_B1,), lambda i: (i,)),
      out_shape=jax.ShapeDtypeStruct((NP,), _F32),
  )(p1, xt_p, W1, b1)


def _t2_body(p_ref, a_ref, o_ref):
  mn = jnp.min(p_ref[0:16], axis=0)
  mx = jnp.max(p_ref[16:32], axis=0)
  a = a_ref[...]
  o_ref[...] = jnp.maximum(a - mn, mx - a)


def _t2(p2, a):
  return pl.pallas_call(
      _t2_body,
      grid=(NP // _B1,),
      in_specs=[
          pl.BlockSpec((32, _B1), lambda i: (0, i)),
          pl.BlockSpec((_B1,), lambda i: (i,)),
      ],
      out_specs=pl.BlockSpec((_B1,), lambda i: (i,)),
      out_shape=jax.ShapeDtypeStruct((NP,), _F32),
  )(p2, a)


_B3 = 1024


def _t3_body(p_ref, t_ref, w2_ref, b2_ref, w3_ref, b3_ref, o_ref):
  tn = jnp.min(p_ref[0:16], axis=0)
  tx = jnp.max(p_ref[16:32], axis=0)
  t = t_ref[...]
  w2 = w2_ref[0]
  b2 = b2_ref[0]
  w3 = w3_ref[0]

  def g(u):
    return jnp.maximum(u[:, None] * w2[None, :] + b2[None, :], 0.0)

  ht = g(t)
  m3 = jnp.maximum(jnp.abs(ht - g(tn)), jnp.abs(ht - g(tx)))
  z = jnp.sum(m3 * w3[None, :], axis=1) + b3_ref[0]
  o_ref[...] = jax.nn.sigmoid(z)


def _t3(p3, m2, W2, b2, W3, b3):
  return pl.pallas_call(
      _t3_body,
      grid=(NP // _B3,),
      in_specs=[
          pl.BlockSpec((32, _B3), lambda i: (0, i)),
          pl.BlockSpec((_B3,), lambda i: (i,)),
          pl.BlockSpec((1, 64), lambda i: (0, 0)),
          pl.BlockSpec((1, 64), lambda i: (0, 0)),
          pl.BlockSpec((1, 64), lambda i: (0, 0)),
          pl.BlockSpec(memory_space=pltpu.SMEM),
      ],
      out_specs=pl.BlockSpec((_B3,), lambda i: (i,)),
      out_shape=jax.ShapeDtypeStruct((NP,), _F32),
  )(p3, m2, W2, b2.reshape(1, 64), W3.reshape(1, 64), b3)


def kernel(x, edge_index, W1, b1, W2, b2, W3, b3):
  xt = x.T
  xt_p = jnp.pad(xt, ((0, 0), (0, NP - N)))
  src = edge_index[0]
  dst = edge_index[1]

  p1 = _edge_scatter8(xt, src, dst).reshape(32, NP)
  a = _t1(p1, xt_p, W1, b1)
  p2 = _edge_scatter1(a, src, dst).reshape(32, NP)
  m2 = _t2(p2, a)
  p3 = _edge_scatter1(m2, src, dst).reshape(32, NP)
  out = _t3(p3, m2, W2, b2, W3, b3)
  return out[:N].reshape(N, 1)


# speedup vs baseline: 4.3573x; 1.4985x over previous
import functools

import jax
import jax.numpy as jnp
from jax import lax
from jax.experimental import pallas as pl
from jax.experimental.pallas import tpu as pltpu
from jax.experimental.pallas import tpu_sc as plsc

N = 50000
E = 800000
D = 8
NP = 51200
NC = 2
NS = 16

_U = 5
_F32 = jnp.float32
_I32 = jnp.int32


def _dyn_gather(v, idx):
  dnums = lax.GatherDimensionNumbers(
      offset_dims=(), collapsed_slice_dims=(0,), start_index_map=(0,))
  return lax.gather(v, idx[:, None], dnums, (1,),
                    mode=lax.GatherScatterMode.PROMISE_IN_BOUNDS)


def _combine(is_min_vec, a, b):
  return jnp.where(is_min_vec, jnp.minimum(a, b), jnp.maximum(a, b))


def _slow_sort_vec(src_buf, dst_buf, src_arr, acc, j, is_min_vec, iota,
                   shift_idx, next_idx, last_mask):
  s = src_buf[pl.ds(j * 16, 16)]
  d = dst_buf[pl.ds(j * 16, 16)]
  v = plsc.load_gather(src_arr, [s])
  res = plsc.sort_key_val(d, v)
  d_s, v_s = res[0], res[1]
  for t, pidx in enumerate(shift_idx):
    pk = _dyn_gather(d_s, pidx)
    pv = _dyn_gather(v_s, pidx)
    ok = (pk == d_s) & (iota >= (1 << t))
    v_s = jnp.where(ok, _combine(is_min_vec, v_s, pv), v_s)
  nk = _dyn_gather(d_s, next_idx)
  is_end = (d_s != nk) | last_mask
  cur = plsc.load_gather(acc, [d_s])
  plsc.store_scatter(acc, [d_s], _combine(is_min_vec, cur, v_s), mask=is_end)


def _edge_scatter_body(src_vals_h, src_h, dst_h, out_h, val_v, acc,
                       sb0, db0, sb1, db1, sem0, sem1,
                       *, n_streams, e_slice, ch, row_of, vrow_of):
  wid = lax.axis_index("s") * NC + lax.axis_index("c")
  kind = (wid // n_streams) % 2
  is_min_vec = (jnp.zeros((16,), _I32) + kind) == 0
  iota = lax.iota(_I32, 16)
  shift_idx = tuple(jnp.maximum(iota - (1 << t), 0) for t in range(4))
  next_idx = jnp.minimum(iota + 1, 15)
  last_mask = iota == 15
  nv = ch // 16

  voff = pl.multiple_of(vrow_of(wid) * N, 8)
  pltpu.sync_copy(src_vals_h.at[pl.ds(voff, N)], val_v)
  pltpu.sync_copy(src_vals_h.at[pl.ds(voff, N)], acc)

  slice_id = wid // (2 * n_streams)
  base = slice_id * e_slice
  nch = e_slice // ch
  pairs = nch // 2

  def start(cidx, sbuf, dbuf, sem):
    off = pl.multiple_of(base + cidx * ch, 8)
    pltpu.async_copy(src_h.at[pl.ds(off, ch)], sbuf, sem)
    pltpu.async_copy(dst_h.at[pl.ds(off, ch)], dbuf, sem)

  def wait(sbuf, dbuf, sem):
    pltpu.make_async_copy(src_h.at[pl.ds(0, ch)], sbuf, sem).wait()
    pltpu.make_async_copy(dst_h.at[pl.ds(0, ch)], dbuf, sem).wait()

  def process(sbuf, dbuf):
    def vec_body(g, bad):
      j0 = g * _U
      ds_ = [dbuf[pl.ds((j0 + u) * 16, 16)] for u in range(_U)]
      ss_ = [sbuf[pl.ds((j0 + u) * 16, 16)] for u in range(_U)]
      vs_ = [plsc.load_gather(val_v, [t]) for t in ss_]
      curs = [plsc.load_gather(acc, [d]) for d in ds_]
      for d, cur, v in zip(ds_, curs, vs_):
        plsc.store_scatter(acc, [d], _combine(is_min_vec, cur, v))
      backs = [plsc.load_gather(acc, [d]) for d in ds_]
      needs = [jnp.where(is_min_vec, bk > v, bk < v)
               for bk, v in zip(backs, vs_)]
      for d, v, need in zip(ds_, vs_, needs):
        plsc.store_scatter(acc, [d], v, mask=need)
      back2s = [plsc.load_gather(acc, [d]) for d in ds_]
      for b2, v in zip(back2s, vs_):
        bad = bad | jnp.where(is_min_vec, b2 > v, b2 < v)
      return bad

    bad = lax.fori_loop(0, nv // _U, vec_body, jnp.zeros((16,), jnp.bool_))

    def cleanup():
      def cb(j, c):
        _slow_sort_vec(sbuf, dbuf, val_v, acc, j, is_min_vec, iota,
                       shift_idx, next_idx, last_mask)
        return c

      lax.fori_loop(0, nv, cb, 0)

    lax.cond(jnp.any(bad), cleanup, lambda: None)

  start(0, sb0, db0, sem0)

  def pair_body(k, c):
    start(2 * k + 1, sb1, db1, sem1)
    wait(sb0, db0, sem0)
    process(sb0, db0)
    start(jnp.minimum(2 * k + 2, nch - 1), sb0, db0, sem0)
    wait(sb1, db1, sem1)
    process(sb1, db1)
    return c

  lax.fori_loop(0, pairs, pair_body, 0)
  wait(sb0, db0, sem0)
  if nch % 2 == 1:
    process(sb0, db0)
  ooff = pl.multiple_of(row_of(wid) * NP, 8)
  pltpu.sync_copy(acc, out_h.at[pl.ds(ooff, N)])


def _edge_scatter8(xt, src, dst):
  ch = 4000
  mesh = plsc.VectorSubcoreMesh(core_axis_name="c", subcore_axis_name="s")
  body = functools.partial(
      _edge_scatter_body,
      n_streams=8, e_slice=E // 2, ch=ch,
      row_of=lambda wid: (wid % 8) * 4 + ((wid // 8) % 2) * 2 + wid // 16,
      vrow_of=lambda wid: wid % 8)

  @functools.partial(
      pl.kernel, mesh=mesh,
      compiler_params=pltpu.CompilerParams(needs_layout_passes=False),
      out_type=jax.ShapeDtypeStruct((32 * NP,), _F32),
      scratch_types=[
          pltpu.VMEM((N,), _F32), pltpu.VMEM((N,), _F32),
          pltpu.VMEM((ch,), _I32), pltpu.VMEM((ch,), _I32),
          pltpu.VMEM((ch,), _I32), pltpu.VMEM((ch,), _I32),
          pltpu.SemaphoreType.DMA, pltpu.SemaphoreType.DMA,
      ])
  def run(xt_h, src_h, dst_h, out_h, val_v, acc, sb0, db0, sb1, db1,
          sem0, sem1):
    body(xt_h, src_h, dst_h, out_h, val_v, acc, sb0, db0, sb1, db1,
         sem0, sem1)

  return run(xt.reshape(-1), src, dst)


def _edge_scatter1(vals, src, dst):
  ch = 2000
  mesh = plsc.VectorSubcoreMesh(core_axis_name="c", subcore_axis_name="s")
  body = functools.partial(
      _edge_scatter_body,
      n_streams=1, e_slice=E // 16, ch=ch,
      row_of=lambda wid: (wid % 2) * 16 + wid // 2,
      vrow_of=lambda wid: 0)

  @functools.partial(
      pl.kernel, mesh=mesh,
      compiler_params=pltpu.CompilerParams(needs_layout_passes=False),
      out_type=jax.ShapeDtypeStruct((32 * NP,), _F32),
      scratch_types=[
          pltpu.VMEM((N,), _F32), pltpu.VMEM((N,), _F32),
          pltpu.VMEM((ch,), _I32), pltpu.VMEM((ch,), _I32),
          pltpu.VMEM((ch,), _I32), pltpu.VMEM((ch,), _I32),
          pltpu.SemaphoreType.DMA, pltpu.SemaphoreType.DMA,
      ])
  def run(vals_h, src_h, dst_h, out_h, val_v, acc, sb0, db0, sb1, db1,
          sem0, sem1):
    body(vals_h, src_h, dst_h, out_h, val_v, acc, sb0, db0, sb1, db1,
         sem0, sem1)

  return run(vals, src, dst)


_B1 = 2048


def _t1_body(p_ref, x_ref, w_ref, b_ref, o_ref):
  acc = jnp.zeros((_B1,), _F32)
  for c in range(D):
    mn = jnp.minimum(p_ref[4 * c + 0], p_ref[4 * c + 1])
    mx = jnp.maximum(p_ref[4 * c + 2], p_ref[4 * c + 3])
    xc = x_ref[c]
    m = jnp.maximum(xc - mn, mx - xc)
    acc = acc + m * w_ref[c, 0]
  o_ref[...] = jnp.maximum(acc + b_ref[0], 0.0)


def _t1(p1, xt_p, W1, b1):
  return pl.pallas_call(
      _t1_body,
      grid=(NP // _B1,),
      in_specs=[
          pl.BlockSpec((32, _B1), lambda i: (0, i)),
          pl.BlockSpec((D, _B1), lambda i: (0, i)),
          pl.BlockSpec(memory_space=pltpu.SMEM),
          pl.BlockSpec(memory_space=pltpu.SMEM),
      ],
      out_specs=pl.BlockSpec((_B1,), lambda i: (i,)),
      out_shape=jax.ShapeDtypeStruct((NP,), _F32),
  )(p1, xt_p, W1, b1)


def _t2_body(p_ref, a_ref, o_ref):
  mn = jnp.min(p_ref[0:16], axis=0)
  mx = jnp.max(p_ref[16:32], axis=0)
  a = a_ref[...]
  o_ref[...] = jnp.maximum(a - mn, mx - a)


def _t2(p2, a):
  return pl.pallas_call(
      _t2_body,
      grid=(NP // _B1,),
      in_specs=[
          pl.BlockSpec((32, _B1), lambda i: (0, i)),
          pl.BlockSpec((_B1,), lambda i: (i,)),
      ],
      out_specs=pl.BlockSpec((_B1,), lambda i: (i,)),
      out_shape=jax.ShapeDtypeStruct((NP,), _F32),
  )(p2, a)


_B3 = 1024


def _t3_body(p_ref, t_ref, w2_ref, b2_ref, w3_ref, b3_ref, o_ref):
  tn = jnp.min(p_ref[0:16], axis=0)
  tx = jnp.max(p_ref[16:32], axis=0)
  t = t_ref[...]
  w2 = w2_ref[0]
  b2 = b2_ref[0]
  w3 = w3_ref[0]

  def g(u):
    return jnp.maximum(u[:, None] * w2[None, :] + b2[None, :], 0.0)

  ht = g(t)
  m3 = jnp.maximum(jnp.abs(ht - g(tn)), jnp.abs(ht - g(tx)))
  z = jnp.sum(m3 * w3[None, :], axis=1) + b3_ref[0]
  o_ref[...] = jax.nn.sigmoid(z)


def _t3(p3, m2, W2, b2, W3, b3):
  return pl.pallas_call(
      _t3_body,
      grid=((N + _B3 - 1) // _B3,),
      in_specs=[
          pl.BlockSpec((32, _B3), lambda i: (0, i)),
          pl.BlockSpec((_B3,), lambda i: (i,)),
          pl.BlockSpec((1, 64), lambda i: (0, 0)),
          pl.BlockSpec((1, 64), lambda i: (0, 0)),
          pl.BlockSpec((1, 64), lambda i: (0, 0)),
          pl.BlockSpec(memory_space=pltpu.SMEM),
      ],
      out_specs=pl.BlockSpec((_B3,), lambda i: (i,)),
      out_shape=jax.ShapeDtypeStruct((N,), _F32),
  )(p3, m2, W2, b2.reshape(1, 64), W3.reshape(1, 64), b3)


def kernel(x, edge_index, W1, b1, W2, b2, W3, b3):
  xt = x.T
  xt_p = jnp.pad(xt, ((0, 0), (0, NP - N)))
  src = edge_index[0]
  dst = edge_index[1]

  p1 = _edge_scatter8(xt, src, dst).reshape(32, NP)
  a = _t1(p1, xt_p, W1, b1)
  p2 = _edge_scatter1(a, src, dst).reshape(32, NP)
  m2 = _t2(p2, a)
  p3 = _edge_scatter1(m2, src, dst).reshape(32, NP)
  out = _t3(p3, m2, W2, b2, W3, b3)
  return out.reshape(N, 1)


# speedup vs baseline: 4.6417x; 1.0653x over previous
import functools

import jax
import jax.numpy as jnp
from jax import lax
from jax.experimental import pallas as pl
from jax.experimental.pallas import tpu as pltpu
from jax.experimental.pallas import tpu_sc as plsc

N = 50000
E = 800000
D = 8
NP = 51200
NC = 2
NS = 16

_U = 5
_F32 = jnp.float32
_I32 = jnp.int32


def _dyn_gather(v, idx):
  dnums = lax.GatherDimensionNumbers(
      offset_dims=(), collapsed_slice_dims=(0,), start_index_map=(0,))
  return lax.gather(v, idx[:, None], dnums, (1,),
                    mode=lax.GatherScatterMode.PROMISE_IN_BOUNDS)


def _combine(is_min_vec, a, b):
  return jnp.where(is_min_vec, jnp.minimum(a, b), jnp.maximum(a, b))


def _slow_sort_vec(src_buf, dst_buf, src_arr, acc, j, is_min_vec, iota,
                   shift_idx, next_idx, last_mask):
  s = src_buf[pl.ds(j * 16, 16)]
  d = dst_buf[pl.ds(j * 16, 16)]
  v = plsc.load_gather(src_arr, [s])
  res = plsc.sort_key_val(d, v)
  d_s, v_s = res[0], res[1]
  for t, pidx in enumerate(shift_idx):
    pk = _dyn_gather(d_s, pidx)
    pv = _dyn_gather(v_s, pidx)
    ok = (pk == d_s) & (iota >= (1 << t))
    v_s = jnp.where(ok, _combine(is_min_vec, v_s, pv), v_s)
  nk = _dyn_gather(d_s, next_idx)
  is_end = (d_s != nk) | last_mask
  cur = plsc.load_gather(acc, [d_s])
  plsc.store_scatter(acc, [d_s], _combine(is_min_vec, cur, v_s), mask=is_end)


def _edge_scatter_body(src_vals_h, src_h, dst_h, out_h, val_v, acc,
                       sb0, db0, sb1, db1, sem0, sem1,
                       *, n_streams, e_slice, ch, row_of, vrow_of):
  wid = lax.axis_index("s") * NC + lax.axis_index("c")
  kind = (wid // n_streams) % 2
  is_min_vec = (jnp.zeros((16,), _I32) + kind) == 0
  iota = lax.iota(_I32, 16)
  shift_idx = tuple(jnp.maximum(iota - (1 << t), 0) for t in range(4))
  next_idx = jnp.minimum(iota + 1, 15)
  last_mask = iota == 15
  nv = ch // 16

  voff = pl.multiple_of(vrow_of(wid) * N, 8)
  pltpu.sync_copy(src_vals_h.at[pl.ds(voff, N)], val_v)
  pltpu.sync_copy(src_vals_h.at[pl.ds(voff, N)], acc)

  slice_id = wid // (2 * n_streams)
  base = slice_id * e_slice
  nch = e_slice // ch
  pairs = nch // 2

  def start(cidx, sbuf, dbuf, sem):
    off = pl.multiple_of(base + cidx * ch, 8)
    pltpu.async_copy(src_h.at[pl.ds(off, ch)], sbuf, sem)
    pltpu.async_copy(dst_h.at[pl.ds(off, ch)], dbuf, sem)

  def wait(sbuf, dbuf, sem):
    pltpu.make_async_copy(src_h.at[pl.ds(0, ch)], sbuf, sem).wait()
    pltpu.make_async_copy(dst_h.at[pl.ds(0, ch)], dbuf, sem).wait()

  def process(sbuf, dbuf):
    def vec_body(g, bad):
      j0 = g * _U
      ds_ = [dbuf[pl.ds((j0 + u) * 16, 16)] for u in range(_U)]
      ss_ = [sbuf[pl.ds((j0 + u) * 16, 16)] for u in range(_U)]
      vs_ = [plsc.load_gather(val_v, [t]) for t in ss_]
      curs = [plsc.load_gather(acc, [d]) for d in ds_]
      for d, cur, v in zip(ds_, curs, vs_):
        plsc.store_scatter(acc, [d], _combine(is_min_vec, cur, v))
      backs = [plsc.load_gather(acc, [d]) for d in ds_]
      needs = [jnp.where(is_min_vec, bk > v, bk < v)
               for bk, v in zip(backs, vs_)]
      for d, v, need in zip(ds_, vs_, needs):
        plsc.store_scatter(acc, [d], v, mask=need)
      back2s = [plsc.load_gather(acc, [d]) for d in ds_]
      for b2, v in zip(back2s, vs_):
        bad = bad | jnp.where(is_min_vec, b2 > v, b2 < v)
      return bad

    bad = lax.fori_loop(0, nv // _U, vec_body, jnp.zeros((16,), jnp.bool_))

    def cleanup():
      def cb(j, c):
        _slow_sort_vec(sbuf, dbuf, val_v, acc, j, is_min_vec, iota,
                       shift_idx, next_idx, last_mask)
        return c

      lax.fori_loop(0, nv, cb, 0)

    lax.cond(jnp.any(bad), cleanup, lambda: None)

  start(0, sb0, db0, sem0)

  def pair_body(k, c):
    start(2 * k + 1, sb1, db1, sem1)
    wait(sb0, db0, sem0)
    process(sb0, db0)
    start(jnp.minimum(2 * k + 2, nch - 1), sb0, db0, sem0)
    wait(sb1, db1, sem1)
    process(sb1, db1)
    return c

  lax.fori_loop(0, pairs, pair_body, 0)
  wait(sb0, db0, sem0)
  if nch % 2 == 1:
    process(sb0, db0)
  ooff = pl.multiple_of(row_of(wid) * NP, 8)
  pltpu.sync_copy(acc, out_h.at[pl.ds(ooff, N)])


def _edge_scatter8(xt, src, dst):
  ch = 4000
  mesh = plsc.VectorSubcoreMesh(core_axis_name="c", subcore_axis_name="s")
  body = functools.partial(
      _edge_scatter_body,
      n_streams=8, e_slice=E // 2, ch=ch,
      row_of=lambda wid: (wid % 8) * 4 + ((wid // 8) % 2) * 2 + wid // 16,
      vrow_of=lambda wid: wid % 8)

  @functools.partial(
      pl.kernel, mesh=mesh,
      compiler_params=pltpu.CompilerParams(needs_layout_passes=False),
      out_type=jax.ShapeDtypeStruct((32 * NP,), _F32),
      scratch_types=[
          pltpu.VMEM((N,), _F32), pltpu.VMEM((N,), _F32),
          pltpu.VMEM((ch,), _I32), pltpu.VMEM((ch,), _I32),
          pltpu.VMEM((ch,), _I32), pltpu.VMEM((ch,), _I32),
          pltpu.SemaphoreType.DMA, pltpu.SemaphoreType.DMA,
      ])
  def run(xt_h, src_h, dst_h, out_h, val_v, acc, sb0, db0, sb1, db1,
          sem0, sem1):
    body(xt_h, src_h, dst_h, out_h, val_v, acc, sb0, db0, sb1, db1,
         sem0, sem1)

  return run(xt.reshape(-1), src, dst)


def _edge_scatter1(vals, src, dst):
  ch = 2000
  mesh = plsc.VectorSubcoreMesh(core_axis_name="c", subcore_axis_name="s")
  body = functools.partial(
      _edge_scatter_body,
      n_streams=1, e_slice=E // 16, ch=ch,
      row_of=lambda wid: (wid % 2) * 16 + wid // 2,
      vrow_of=lambda wid: 0)

  @functools.partial(
      pl.kernel, mesh=mesh,
      compiler_params=pltpu.CompilerParams(needs_layout_passes=False),
      out_type=jax.ShapeDtypeStruct((32 * NP,), _F32),
      scratch_types=[
          pltpu.VMEM((N,), _F32), pltpu.VMEM((N,), _F32),
          pltpu.VMEM((ch,), _I32), pltpu.VMEM((ch,), _I32),
          pltpu.VMEM((ch,), _I32), pltpu.VMEM((ch,), _I32),
          pltpu.SemaphoreType.DMA, pltpu.SemaphoreType.DMA,
      ])
  def run(vals_h, src_h, dst_h, out_h, val_v, acc, sb0, db0, sb1, db1,
          sem0, sem1):
    body(vals_h, src_h, dst_h, out_h, val_v, acc, sb0, db0, sb1, db1,
         sem0, sem1)

  return run(vals, src, dst)


_B1 = 2048


def _t1_body(p_ref, x_ref, w_ref, b_ref, o_ref):
  acc = jnp.zeros((_B1,), _F32)
  for c in range(D):
    mn = jnp.minimum(p_ref[4 * c + 0], p_ref[4 * c + 1])
    mx = jnp.maximum(p_ref[4 * c + 2], p_ref[4 * c + 3])
    xc = x_ref[c]
    m = jnp.maximum(xc - mn, mx - xc)
    acc = acc + m * w_ref[c, 0]
  o_ref[...] = jnp.maximum(acc + b_ref[0], 0.0)


def _t1(p1, xt_p, W1, b1):
  return pl.pallas_call(
      _t1_body,
      grid=(NP // _B1,),
      in_specs=[
          pl.BlockSpec((32, _B1), lambda i: (0, i)),
          pl.BlockSpec((D, _B1), lambda i: (0, i)),
          pl.BlockSpec(memory_space=pltpu.SMEM),
          pl.BlockSpec(memory_space=pltpu.SMEM),
      ],
      out_specs=pl.BlockSpec((_B1,), lambda i: (i,)),
      out_shape=jax.ShapeDtypeStruct((NP,), _F32),
  )(p1, xt_p, W1, b1)


def _t2_body(p_ref, a_ref, o_ref):
  mn = jnp.min(p_ref[0:16], axis=0)
  mx = jnp.max(p_ref[16:32], axis=0)
  a = a_ref[...]
  o_ref[...] = jnp.maximum(a - mn, mx - a)


def _t2(p2, a):
  return pl.pallas_call(
      _t2_body,
      grid=(NP // _B1,),
      in_specs=[
          pl.BlockSpec((32, _B1), lambda i: (0, i)),
          pl.BlockSpec((_B1,), lambda i: (i,)),
      ],
      out_specs=pl.BlockSpec((_B1,), lambda i: (i,)),
      out_shape=jax.ShapeDtypeStruct((NP,), _F32),
  )(p2, a)


_B3 = 1024


def _t3_body(p_ref, t_ref, w2_ref, b2_ref, w3_ref, b3_ref, o_ref):
  tn = jnp.min(p_ref[0:16], axis=0)
  tx = jnp.max(p_ref[16:32], axis=0)
  t = t_ref[...]
  w2 = w2_ref[0]
  b2 = b2_ref[0]
  w3 = w3_ref[0]

  def g(u):
    return jnp.maximum(u[:, None] * w2[None, :] + b2[None, :], 0.0)

  ht = g(t)
  m3 = jnp.maximum(jnp.abs(ht - g(tn)), jnp.abs(ht - g(tx)))
  z = jnp.sum(m3 * w3[None, :], axis=1) + b3_ref[0]
  o_ref[...] = jax.nn.sigmoid(z)


def _t3z_body(p_ref, t_ref, w2_ref, w3_ref, b3_ref, o_ref):
  tn = jnp.min(p_ref[0:16], axis=0)
  tx = jnp.max(p_ref[16:32], axis=0)
  t = t_ref[...]
  w2 = w2_ref[0]
  w3 = w3_ref[0]
  p_pos = jnp.sum(jnp.where(w2 > 0.0, w2 * w3, 0.0))
  p_neg = jnp.sum(jnp.where(w2 < 0.0, -w2 * w3, 0.0))
  tp = jnp.maximum(t, 0.0)
  a_pos = jnp.maximum(tp - jnp.maximum(tn, 0.0), jnp.maximum(tx, 0.0) - tp)
  tr = jnp.maximum(-t, 0.0)
  a_neg = jnp.maximum(jnp.maximum(-tn, 0.0) - tr, tr - jnp.maximum(-tx, 0.0))
  o_ref[...] = jax.nn.sigmoid(a_pos * p_pos + a_neg * p_neg + b3_ref[0])


def _t3z(p3, m2, W2, W3, b3):
  return pl.pallas_call(
      _t3z_body,
      grid=((N + _B3 - 1) // _B3,),
      in_specs=[
          pl.BlockSpec((32, _B3), lambda i: (0, i)),
          pl.BlockSpec((_B3,), lambda i: (i,)),
          pl.BlockSpec((1, 64), lambda i: (0, 0)),
          pl.BlockSpec((1, 64), lambda i: (0, 0)),
          pl.BlockSpec(memory_space=pltpu.SMEM),
      ],
      out_specs=pl.BlockSpec((_B3,), lambda i: (i,)),
      out_shape=jax.ShapeDtypeStruct((N,), _F32),
  )(p3, m2, W2, W3.reshape(1, 64), b3)


def _t3(p3, m2, W2, b2, W3, b3):
  return pl.pallas_call(
      _t3_body,
      grid=((N + _B3 - 1) // _B3,),
      in_specs=[
          pl.BlockSpec((32, _B3), lambda i: (0, i)),
          pl.BlockSpec((_B3,), lambda i: (i,)),
          pl.BlockSpec((1, 64), lambda i: (0, 0)),
          pl.BlockSpec((1, 64), lambda i: (0, 0)),
          pl.BlockSpec((1, 64), lambda i: (0, 0)),
          pl.BlockSpec(memory_space=pltpu.SMEM),
      ],
      out_specs=pl.BlockSpec((_B3,), lambda i: (i,)),
      out_shape=jax.ShapeDtypeStruct((N,), _F32),
  )(p3, m2, W2, b2.reshape(1, 64), W3.reshape(1, 64), b3)


def kernel(x, edge_index, W1, b1, W2, b2, W3, b3):
  xt = x.T
  xt_p = jnp.pad(xt, ((0, 0), (0, NP - N)))
  src = edge_index[0]
  dst = edge_index[1]

  p1 = _edge_scatter8(xt, src, dst).reshape(32, NP)
  a = _t1(p1, xt_p, W1, b1)
  p2 = _edge_scatter1(a, src, dst).reshape(32, NP)
  m2 = _t2(p2, a)
  p3 = _edge_scatter1(m2, src, dst).reshape(32, NP)
  out = lax.cond(jnp.all(b2 == 0.0),
                 lambda: _t3z(p3, m2, W2, W3, b3),
                 lambda: _t3(p3, m2, W2, b2, W3, b3))
  return out.reshape(N, 1)
